# baseline (speedup 1.0000x reference)
import jax
import jax.numpy as jnp
from jax import lax
from jax.experimental import pallas as pl
from jax.experimental.pallas import tpu as pltpu
from jax.experimental.pallas import tpu_sc as plsc

NN = 100000
NE = 6400000
NS = 100000
NC, NT = 2, 16
NW = NC * NT
EPW = NE // NW
B = 2000
NB = EPW // B
NV = B // 16
PAD = 100096
STRIPE = PAD // NT
CHUNK = 448
WSPAN = 896
WROW = WSPAN + 1


def _sc_segsum(x, ptrs, csr):
    mesh = plsc.VectorSubcoreMesh(core_axis_name="c", subcore_axis_name="s",
                                  num_cores=NC, num_subcores=NT)

    def body(x_hbm, ptrs_hbm, csr_hbm, out_hbm, xloc,
             pb0, pb1, cb0, cb1, lacc, iota_ids, ids_stage, val_stage,
             acc, sp0, sp1, sq0, sq1):
        cid = lax.axis_index("c")
        sid = lax.axis_index("s")
        wid = sid * NC + cid
        pbs, cbs = (pb0, pb1), (cb0, cb1)
        sps, sqs = (sp0, sp1), (sq0, sq1)

        iota = lax.iota(jnp.int32, 16)
        lane15 = iota == 15
        zerosf = jnp.zeros((16,), jnp.float32)
        dummy_ids = jnp.full((16,), PAD - 1, jnp.int32)

        def issue_in(slot, b):
            base = wid * EPW + b * B
            pltpu.async_copy(ptrs_hbm.at[pl.ds(base, B)], pbs[slot], sps[slot])
            pltpu.async_copy(csr_hbm.at[pl.ds(base, B)], cbs[slot], sqs[slot])

        def wait_in(slot):
            pltpu.make_async_copy(ptrs_hbm.at[pl.ds(0, B)], pbs[slot],
                                  sps[slot]).wait()
            pltpu.make_async_copy(csr_hbm.at[pl.ds(0, B)],
                                  cbs[slot], sqs[slot]).wait()

        issue_in(0, 0)
        pltpu.sync_copy(x_hbm, xloc)

        def fill_iota(k, c):
            iota_ids[pl.ds(16 * k, 16)] = iota + 16 * k
            return c

        lax.fori_loop(0, CHUNK // 16, fill_iota, 0)

        def zla(k, c):
            lacc[pl.ds(16 * k, 16)] = zerosf
            return c

        lax.fori_loop(0, (16 * WROW) // 16, zla, 0)

        def zvs(k, c):
            val_stage[pl.ds(16 * k, 16)] = zerosf
            return c

        lax.fori_loop(0, CHUNK // 16, zvs, 0)
        for k in range(STRIPE // CHUNK):
            pltpu.sync_copy(val_stage.at[pl.ds(0, CHUNK)],
                            acc.at[pl.ds(sid * STRIPE + k * CHUNK, CHUNK)])
        rem = STRIPE % CHUNK
        pltpu.sync_copy(val_stage.at[pl.ds(0, rem)],
                        acc.at[pl.ds(sid * STRIPE + STRIPE - rem, rem)])
        plsc.subcore_barrier()

        def drain_window(wbase, wmax):
            span = wmax - wbase
            nch = (span + CHUNK - 1) // CHUNK

            def chunk(ci, c):
                boff = wbase + ci * CHUNK

                def mk(k, c2):
                    col = ci * CHUNK + 16 * k
                    s = lacc[pl.ds(col, 16)]
                    for r in range(1, 16):
                        s = s + lacc[pl.ds(r * WROW + col, 16)]
                    val_stage[pl.ds(16 * k, 16)] = s
                    ids_stage[pl.ds(16 * k, 16)] = jnp.minimum(
                        iota_ids[pl.ds(16 * k, 16)] + boff, PAD - 1)
                    return c2

                lax.fori_loop(0, CHUNK // 16, mk, 0)
                pltpu.sync_copy(val_stage, acc.at[ids_stage], add=True)
                return c

            lax.fori_loop(0, nch, chunk, 0)

            def rz(k, c):
                for r in range(16):
                    lacc[pl.ds(r * WROW + 16 * k, 16)] = zerosf
                return c

            lax.fori_loop(0, nch * (CHUNK // 16), rz, 0)

        def accum_block(slot, wbase):
            base_off = iota * WROW - wbase

            @plsc.parallel_loop(0, NV, unroll=16)
            def inner(j):
                ids = cbs[slot][pl.ds(j * 16, 16)]
                ptr = pbs[slot][pl.ds(j * 16, 16)]
                v = plsc.load_gather(xloc, [ptr])
                plsc.addupdate_scatter(lacc, [ids + base_off], v)

        def fallback_block(slot):
            for k in range(-(-B // CHUNK)):
                cnt = min(CHUNK, B - k * CHUNK)

                def mv(t, c):
                    pos = k * CHUNK + t * 16
                    ids_stage[pl.ds(t * 16, 16)] = cbs[slot][pl.ds(pos, 16)]
                    ptr = pbs[slot][pl.ds(pos, 16)]
                    val_stage[pl.ds(t * 16, 16)] = plsc.load_gather(xloc, [ptr])
                    return c

                lax.fori_loop(0, cnt // 16, mv, 0)
                for t in range(cnt // 16, CHUNK // 16):
                    ids_stage[pl.ds(t * 16, 16)] = dummy_ids
                    val_stage[pl.ds(t * 16, 16)] = zerosf
                pltpu.sync_copy(val_stage, acc.at[ids_stage], add=True)

        def do_block(slot, b, carry):
            wbase, wmax = carry
            newmin = cbs[slot][pl.ds(0, 16)][0]
            newmax = cbs[slot][pl.ds(B - 16, 16)][15] + 1
            need_rebase = jnp.logical_or(wbase < 0, newmax - wbase > WSPAN)

            @pl.when(jnp.logical_and(need_rebase, wbase >= 0))
            def _():
                drain_window(wbase, wmax)

            wbase2 = jnp.where(need_rebase, newmin, wbase)
            use_window = newmax - wbase2 <= WSPAN

            @pl.when(use_window)
            def _():
                accum_block(slot, wbase2)

            @pl.when(jnp.logical_not(use_window))
            def _():
                fallback_block(slot)

            wbase3 = jnp.where(use_window, wbase2, jnp.int32(-1))
            wmax3 = jnp.where(use_window, newmax, jnp.int32(0))
            return (wbase3, wmax3)

        def pair(i, carry):
            for phase in range(2):
                slot = phase
                b = 2 * i + phase
                wait_in(slot)

                @pl.when(b + 1 < NB)
                def _():
                    issue_in(1 - slot, b + 1)

                carry = do_block(slot, b, carry)
            return carry

        wbase, wmax = lax.fori_loop(0, NB // 2, pair,
                                    (jnp.int32(-1), jnp.int32(0)))

        @pl.when(wbase >= 0)
        def _():
            drain_window(wbase, wmax)

        plsc.subcore_barrier()

        pltpu.sync_copy(acc.at[pl.ds(sid * STRIPE, STRIPE)],
                        xloc.at[pl.ds(0, STRIPE)])
        pltpu.sync_copy(xloc.at[pl.ds(0, STRIPE)],
                        out_hbm.at[pl.ds(cid * PAD + sid * STRIPE, STRIPE)])

    return pl.kernel(
        body,
        out_type=jax.ShapeDtypeStruct((NC * PAD,), jnp.float32),
        mesh=mesh,
        compiler_params=pltpu.CompilerParams(needs_layout_passes=False),
        scratch_types=(
            [pltpu.VMEM((NN,), jnp.float32)]
            + [pltpu.VMEM((B,), jnp.int32) for _ in range(2)]
            + [pltpu.VMEM((B,), jnp.int32) for _ in range(2)]
            + [pltpu.VMEM((16 * WROW,), jnp.float32)]
            + [pltpu.VMEM((CHUNK,), jnp.int32)]
            + [pltpu.VMEM((CHUNK,), jnp.int32)]
            + [pltpu.VMEM((CHUNK,), jnp.float32)]
            + [pltpu.VMEM_SHARED((PAD,), jnp.float32)]
            + [pltpu.SemaphoreType.DMA for _ in range(4)]
        ),
    )(x, ptrs, csr)


def _tc_add(a_ref, b_ref, o_ref):
    o_ref[...] = a_ref[...] + b_ref[...]


def kernel(x, ptrs, csr):
    parts = _sc_segsum(x, ptrs, csr)
    a = parts[:PAD].reshape(PAD // 128, 128)
    b = parts[PAD:].reshape(PAD // 128, 128)
    out = pl.pallas_call(
        _tc_add,
        out_shape=jax.ShapeDtypeStruct((PAD // 128, 128), jnp.float32),
    )(a, b)
    return out.reshape(-1)[:NS]


# speedup vs baseline: 1.0453x; 1.0453x over previous
import jax
import jax.numpy as jnp
from jax import lax
from jax.experimental import pallas as pl
from jax.experimental.pallas import tpu as pltpu
from jax.experimental.pallas import tpu_sc as plsc

NN = 100000
NE = 6400000
NS = 100000
NC, NT = 2, 16
NW = NC * NT
EPW = NE // NW
B = 2000
NB = EPW // B
NV = B // 16
PAD = 100096
STRIPE = PAD // NT
CHUNK = 512
WSPAN = 14336


def _sc_segsum(x, ptrs, csr):
    mesh = plsc.VectorSubcoreMesh(core_axis_name="c", subcore_axis_name="s",
                                  num_cores=NC, num_subcores=NT)

    def body(x_hbm, ptrs_hbm, csr_hbm, out_hbm, xloc,
             pb0, pb1, cb0, cb1, lacc, iota_ids, ids_stage, val_stage,
             acc, sp0, sp1, sq0, sq1):
        cid = lax.axis_index("c")
        sid = lax.axis_index("s")
        wid = sid * NC + cid
        pbs, cbs = (pb0, pb1), (cb0, cb1)
        sps, sqs = (sp0, sp1), (sq0, sq1)

        iota = lax.iota(jnp.int32, 16)
        lane15 = iota == 15
        zerosf = jnp.zeros((16,), jnp.float32)
        dummy_ids = jnp.full((16,), PAD - 1, jnp.int32)

        def issue_in(slot, b):
            base = wid * EPW + b * B
            pltpu.async_copy(ptrs_hbm.at[pl.ds(base, B)], pbs[slot], sps[slot])
            pltpu.async_copy(csr_hbm.at[pl.ds(base, B)],
                             cbs[slot].at[pl.ds(0, B)], sqs[slot])

        def wait_in(slot):
            pltpu.make_async_copy(ptrs_hbm.at[pl.ds(0, B)], pbs[slot],
                                  sps[slot]).wait()
            pltpu.make_async_copy(csr_hbm.at[pl.ds(0, B)],
                                  cbs[slot].at[pl.ds(0, B)], sqs[slot]).wait()

        issue_in(0, 0)
        pltpu.sync_copy(x_hbm, xloc)

        def fill_iota(k, c):
            iota_ids[pl.ds(16 * k, 16)] = iota + 16 * k
            return c

        lax.fori_loop(0, CHUNK // 16, fill_iota, 0)

        def zla(k, c):
            lacc[pl.ds(16 * k, 16)] = zerosf
            return c

        lax.fori_loop(0, WSPAN // 16, zla, 0)

        def zvs(k, c):
            val_stage[pl.ds(16 * k, 16)] = zerosf
            return c

        lax.fori_loop(0, CHUNK // 16, zvs, 0)
        for k in range(STRIPE // CHUNK):
            pltpu.sync_copy(val_stage.at[pl.ds(0, CHUNK)],
                            acc.at[pl.ds(sid * STRIPE + k * CHUNK, CHUNK)])
        rem = STRIPE % CHUNK
        pltpu.sync_copy(val_stage.at[pl.ds(0, rem)],
                        acc.at[pl.ds(sid * STRIPE + STRIPE - rem, rem)])
        plsc.subcore_barrier()

        def drain_window(wbase, wmax):
            span = wmax - wbase
            nch = (span + CHUNK - 1) // CHUNK

            def chunk(ci, c):
                boff = wbase + ci * CHUNK

                def mk(k, c2):
                    ids_stage[pl.ds(16 * k, 16)] = jnp.minimum(
                        iota_ids[pl.ds(16 * k, 16)] + boff, PAD - 1)
                    return c2

                lax.fori_loop(0, CHUNK // 16, mk, 0)
                pltpu.sync_copy(lacc.at[pl.ds(ci * CHUNK, CHUNK)],
                                acc.at[ids_stage], add=True)
                return c

            lax.fori_loop(0, nch, chunk, 0)

            def rz(k, c):
                lacc[pl.ds(16 * k, 16)] = zerosf
                return c

            lax.fori_loop(0, nch * (CHUNK // 16), rz, 0)

        def accum_block(slot, wbase):
            @plsc.parallel_loop(0, NV, unroll=8)
            def inner(j):
                ids = cbs[slot][pl.ds(j * 16, 16)]
                ids_nx = cbs[slot][pl.ds(j * 16 + 1, 16)]
                ptr = pbs[slot][pl.ds(j * 16, 16)]
                v = plsc.load_gather(xloc, [ptr])
                cum = plsc.cumsum(v)
                neq = ids != ids_nx
                addm = jnp.logical_or(neq, lane15)
                subm = jnp.logical_and(neq, jnp.logical_not(lane15))
                plsc.addupdate_scatter(lacc, [ids - wbase], cum, mask=addm)
                plsc.addupdate_scatter(lacc, [ids_nx - wbase], -cum, mask=subm)

        def fallback_block(slot):
            for k in range(4):
                cnt = min(CHUNK, B - k * CHUNK)

                def mv(t, c):
                    pos = k * CHUNK + t * 16
                    ids_stage[pl.ds(t * 16, 16)] = cbs[slot][pl.ds(pos, 16)]
                    ptr = pbs[slot][pl.ds(pos, 16)]
                    val_stage[pl.ds(t * 16, 16)] = plsc.load_gather(xloc, [ptr])
                    return c

                lax.fori_loop(0, cnt // 16, mv, 0)
                for t in range(cnt // 16, CHUNK // 16):
                    ids_stage[pl.ds(t * 16, 16)] = dummy_ids
                    val_stage[pl.ds(t * 16, 16)] = zerosf
                pltpu.sync_copy(val_stage, acc.at[ids_stage], add=True)

        def do_block(slot, b, carry):
            wbase, wmax = carry
            newmin = cbs[slot][pl.ds(0, 16)][0]
            newmax = cbs[slot][pl.ds(B - 16, 16)][15] + 1
            need_rebase = jnp.logical_or(wbase < 0, newmax - wbase > WSPAN)

            @pl.when(jnp.logical_and(need_rebase, wbase >= 0))
            def _():
                drain_window(wbase, wmax)

            wbase2 = jnp.where(need_rebase, newmin, wbase)
            use_window = newmax - wbase2 <= WSPAN

            @pl.when(use_window)
            def _():
                accum_block(slot, wbase2)

            @pl.when(jnp.logical_not(use_window))
            def _():
                fallback_block(slot)

            wbase3 = jnp.where(use_window, wbase2, jnp.int32(-1))
            wmax3 = jnp.where(use_window, newmax, jnp.int32(0))
            return (wbase3, wmax3)

        def pair(i, carry):
            for phase in range(2):
                slot = phase
                b = 2 * i + phase
                wait_in(slot)

                @pl.when(b + 1 < NB)
                def _():
                    issue_in(1 - slot, b + 1)

                carry = do_block(slot, b, carry)
            return carry

        wbase, wmax = lax.fori_loop(0, NB // 2, pair,
                                    (jnp.int32(-1), jnp.int32(0)))

        @pl.when(wbase >= 0)
        def _():
            drain_window(wbase, wmax)

        plsc.subcore_barrier()

        pltpu.sync_copy(acc.at[pl.ds(sid * STRIPE, STRIPE)],
                        xloc.at[pl.ds(0, STRIPE)])
        pltpu.sync_copy(xloc.at[pl.ds(0, STRIPE)],
                        out_hbm.at[pl.ds(cid * PAD + sid * STRIPE, STRIPE)])

    return pl.kernel(
        body,
        out_type=jax.ShapeDtypeStruct((NC * PAD,), jnp.float32),
        mesh=mesh,
        compiler_params=pltpu.CompilerParams(needs_layout_passes=False),
        scratch_types=(
            [pltpu.VMEM((NN,), jnp.float32)]
            + [pltpu.VMEM((B,), jnp.int32) for _ in range(2)]
            + [pltpu.VMEM((B + 16,), jnp.int32) for _ in range(2)]
            + [pltpu.VMEM((WSPAN,), jnp.float32)]
            + [pltpu.VMEM((CHUNK,), jnp.int32)]
            + [pltpu.VMEM((CHUNK,), jnp.int32)]
            + [pltpu.VMEM((CHUNK,), jnp.float32)]
            + [pltpu.VMEM_SHARED((PAD,), jnp.float32)]
            + [pltpu.SemaphoreType.DMA for _ in range(4)]
        ),
    )(x, ptrs, csr)


def _tc_add(a_ref, b_ref, o_ref):
    o_ref[...] = a_ref[...] + b_ref[...]


def kernel(x, ptrs, csr):
    parts = _sc_segsum(x, ptrs, csr)
    a = parts[:PAD].reshape(PAD // 128, 128)
    b = parts[PAD:].reshape(PAD // 128, 128)
    out = pl.pallas_call(
        _tc_add,
        out_shape=jax.ShapeDtypeStruct((PAD // 128, 128), jnp.float32),
    )(a, b)
    return out.reshape(-1)[:NS]


# speedup vs baseline: 1.0456x; 1.0002x over previous
import jax
import jax.numpy as jnp
from jax import lax
from jax.experimental import pallas as pl
from jax.experimental.pallas import tpu as pltpu
from jax.experimental.pallas import tpu_sc as plsc

NN = 100000
NE = 6400000
NS = 100000
NC, NT = 2, 16
NW = NC * NT
EPW = NE // NW
B = 2000
NB = EPW // B
NV = B // 16
PAD = 100096
STRIPE = PAD // NT
CHUNK = 512
WSPAN = 14336


def _sc_segsum(x, ptrs, csr):
    mesh = plsc.VectorSubcoreMesh(core_axis_name="c", subcore_axis_name="s",
                                  num_cores=NC, num_subcores=NT)

    def body(x_hbm, ptrs_hbm, csr_hbm, out_hbm, xloc,
             pb0, pb1, cb0, cb1, lacc, iota_ids, ids_stage, val_stage,
             acc, sp0, sp1, sq0, sq1):
        cid = lax.axis_index("c")
        sid = lax.axis_index("s")
        wid = sid * NC + cid
        pbs, cbs = (pb0, pb1), (cb0, cb1)
        sps, sqs = (sp0, sp1), (sq0, sq1)

        iota = lax.iota(jnp.int32, 16)
        lane15 = iota == 15
        zerosf = jnp.zeros((16,), jnp.float32)
        dummy_ids = jnp.full((16,), PAD - 1, jnp.int32)

        def issue_in(slot, b):
            base = wid * EPW + b * B
            pltpu.async_copy(ptrs_hbm.at[pl.ds(base, B)], pbs[slot], sps[slot])
            pltpu.async_copy(csr_hbm.at[pl.ds(base, B)],
                             cbs[slot].at[pl.ds(0, B)], sqs[slot])

        def wait_in(slot):
            pltpu.make_async_copy(ptrs_hbm.at[pl.ds(0, B)], pbs[slot],
                                  sps[slot]).wait()
            pltpu.make_async_copy(csr_hbm.at[pl.ds(0, B)],
                                  cbs[slot].at[pl.ds(0, B)], sqs[slot]).wait()

        issue_in(0, 0)
        pltpu.sync_copy(x_hbm, xloc)

        def fill_iota(k, c):
            iota_ids[pl.ds(16 * k, 16)] = iota + 16 * k
            return c

        lax.fori_loop(0, CHUNK // 16, fill_iota, 0)

        def zla(k, c):
            lacc[pl.ds(16 * k, 16)] = zerosf
            return c

        lax.fori_loop(0, WSPAN // 16, zla, 0)

        def zvs(k, c):
            val_stage[pl.ds(16 * k, 16)] = zerosf
            return c

        lax.fori_loop(0, CHUNK // 16, zvs, 0)
        for k in range(STRIPE // CHUNK):
            pltpu.sync_copy(val_stage.at[pl.ds(0, CHUNK)],
                            acc.at[pl.ds(sid * STRIPE + k * CHUNK, CHUNK)])
        rem = STRIPE % CHUNK
        pltpu.sync_copy(val_stage.at[pl.ds(0, rem)],
                        acc.at[pl.ds(sid * STRIPE + STRIPE - rem, rem)])
        plsc.subcore_barrier()

        def drain_window(wbase, wmax):
            span = wmax - wbase
            nch = (span + CHUNK - 1) // CHUNK

            def chunk(ci, c):
                boff = wbase + ci * CHUNK

                def mk(k, c2):
                    ids_stage[pl.ds(16 * k, 16)] = jnp.minimum(
                        iota_ids[pl.ds(16 * k, 16)] + boff, PAD - 1)
                    return c2

                lax.fori_loop(0, CHUNK // 16, mk, 0)
                pltpu.sync_copy(lacc.at[pl.ds(ci * CHUNK, CHUNK)],
                                acc.at[ids_stage], add=True)
                return c

            lax.fori_loop(0, nch, chunk, 0)

            def rz(k, c):
                lacc[pl.ds(16 * k, 16)] = zerosf
                return c

            lax.fori_loop(0, nch * (CHUNK // 16), rz, 0)

        def accum_block(slot, wbase):
            shift_idx = jnp.minimum(iota + 1, 15)

            @plsc.parallel_loop(0, NV, unroll=8)
            def inner(j):
                ids = cbs[slot][pl.ds(j * 16, 16)]
                ids_nx = ids.at[shift_idx].get(mode="promise_in_bounds")
                ptr = pbs[slot][pl.ds(j * 16, 16)]
                v = plsc.load_gather(xloc, [ptr])
                cum = plsc.cumsum(v)
                neq = ids != ids_nx
                addm = jnp.logical_or(neq, lane15)
                subm = jnp.logical_and(neq, jnp.logical_not(lane15))
                plsc.addupdate_scatter(lacc, [ids - wbase], cum, mask=addm)
                plsc.addupdate_scatter(lacc, [ids_nx - wbase], -cum, mask=subm)

        def fallback_block(slot):
            for k in range(4):
                cnt = min(CHUNK, B - k * CHUNK)

                def mv(t, c):
                    pos = k * CHUNK + t * 16
                    ids_stage[pl.ds(t * 16, 16)] = cbs[slot][pl.ds(pos, 16)]
                    ptr = pbs[slot][pl.ds(pos, 16)]
                    val_stage[pl.ds(t * 16, 16)] = plsc.load_gather(xloc, [ptr])
                    return c

                lax.fori_loop(0, cnt // 16, mv, 0)
                for t in range(cnt // 16, CHUNK // 16):
                    ids_stage[pl.ds(t * 16, 16)] = dummy_ids
                    val_stage[pl.ds(t * 16, 16)] = zerosf
                pltpu.sync_copy(val_stage, acc.at[ids_stage], add=True)

        def do_block(slot, b, carry):
            wbase, wmax = carry
            newmin = cbs[slot][pl.ds(0, 16)][0]
            newmax = cbs[slot][pl.ds(B - 16, 16)][15] + 1
            need_rebase = jnp.logical_or(wbase < 0, newmax - wbase > WSPAN)

            @pl.when(jnp.logical_and(need_rebase, wbase >= 0))
            def _():
                drain_window(wbase, wmax)

            wbase2 = jnp.where(need_rebase, newmin, wbase)
            use_window = newmax - wbase2 <= WSPAN

            @pl.when(use_window)
            def _():
                accum_block(slot, wbase2)

            @pl.when(jnp.logical_not(use_window))
            def _():
                fallback_block(slot)

            wbase3 = jnp.where(use_window, wbase2, jnp.int32(-1))
            wmax3 = jnp.where(use_window, newmax, jnp.int32(0))
            return (wbase3, wmax3)

        def pair(i, carry):
            for phase in range(2):
                slot = phase
                b = 2 * i + phase
                wait_in(slot)

                @pl.when(b + 1 < NB)
                def _():
                    issue_in(1 - slot, b + 1)

                carry = do_block(slot, b, carry)
            return carry

        wbase, wmax = lax.fori_loop(0, NB // 2, pair,
                                    (jnp.int32(-1), jnp.int32(0)))

        @pl.when(wbase >= 0)
        def _():
            drain_window(wbase, wmax)

        plsc.subcore_barrier()

        pltpu.sync_copy(acc.at[pl.ds(sid * STRIPE, STRIPE)],
                        xloc.at[pl.ds(0, STRIPE)])
        pltpu.sync_copy(xloc.at[pl.ds(0, STRIPE)],
                        out_hbm.at[pl.ds(cid * PAD + sid * STRIPE, STRIPE)])

    return pl.kernel(
        body,
        out_type=jax.ShapeDtypeStruct((NC * PAD,), jnp.float32),
        mesh=mesh,
        compiler_params=pltpu.CompilerParams(needs_layout_passes=False),
        scratch_types=(
            [pltpu.VMEM((NN,), jnp.float32)]
            + [pltpu.VMEM((B,), jnp.int32) for _ in range(2)]
            + [pltpu.VMEM((B + 16,), jnp.int32) for _ in range(2)]
            + [pltpu.VMEM((WSPAN,), jnp.float32)]
            + [pltpu.VMEM((CHUNK,), jnp.int32)]
            + [pltpu.VMEM((CHUNK,), jnp.int32)]
            + [pltpu.VMEM((CHUNK,), jnp.float32)]
            + [pltpu.VMEM_SHARED((PAD,), jnp.float32)]
            + [pltpu.SemaphoreType.DMA for _ in range(4)]
        ),
    )(x, ptrs, csr)


def _tc_add(a_ref, b_ref, o_ref):
    o_ref[...] = a_ref[...] + b_ref[...]


def kernel(x, ptrs, csr):
    parts = _sc_segsum(x, ptrs, csr)
    a = parts[:PAD].reshape(PAD // 128, 128)
    b = parts[PAD:].reshape(PAD // 128, 128)
    out = pl.pallas_call(
        _tc_add,
        out_shape=jax.ShapeDtypeStruct((PAD // 128, 128), jnp.float32),
    )(a, b)
    return out.reshape(-1)[:NS]
